# Initial kernel scaffold; baseline (speedup 1.0000x reference)
import jax
import jax.numpy as jnp
from jax.experimental import pallas as pl


def kernel(z, W_emb):
    raise NotImplementedError("write your pallas kernel here")


# speedup vs baseline: 1.0580x; 1.0580x over previous
import jax
import jax.numpy as jnp
from jax.experimental import pallas as pl
from jax.experimental.pallas import tpu as pltpu
from jax.experimental.pallas import tpu_sc as plsc

_K = 8192
_D = 256
_HW = 1024
_B = 8
_KCHUNK = 1024


def _argmin_body(zh_ref, z_ref, wh_ref, w_ref, ids_ref):
    zbh = zh_ref[0]
    zb = z_ref[0]
    z_sq = jnp.sum(zb * zb, axis=0, keepdims=True)
    run_min = jnp.full((1, _HW), jnp.inf, jnp.float32)
    run_idx = jnp.zeros((1, _HW), jnp.int32)
    for c in range(_K // _KCHUNK):
        wc = w_ref[pl.ds(c * _KCHUNK, _KCHUNK), :]
        w_sq = jnp.sum(wc * wc, axis=1, keepdims=True)
        s = jax.lax.dot_general(
            wh_ref[pl.ds(c * _KCHUNK, _KCHUNK), :], zbh,
            (((1,), (0,)), ((), ())),
            preferred_element_type=jnp.float32)
        dist = ((z_sq - 2.0 * s) + w_sq) * jnp.float32(0.00390625)
        cmin = jnp.min(dist, axis=0, keepdims=True)
        iota = jax.lax.broadcasted_iota(jnp.int32, (_KCHUNK, _HW), 0)
        cidx = jnp.min(jnp.where(dist == cmin, iota, jnp.int32(2**30)),
                       axis=0, keepdims=True) + c * _KCHUNK
        better = cmin < run_min
        run_idx = jnp.where(better, cidx, run_idx)
        run_min = jnp.minimum(run_min, cmin).astype(jnp.bfloat16).astype(
            jnp.float32)
    ids_ref[0] = run_idx


def _token_ids(zrh, zr, W_h, W_emb):
    return pl.pallas_call(
        _argmin_body,
        grid=(_B,),
        in_specs=[
            pl.BlockSpec((1, _D, _HW), lambda b: (b, 0, 0)),
            pl.BlockSpec((1, _D, _HW), lambda b: (b, 0, 0)),
            pl.BlockSpec((_K, _D), lambda b: (0, 0)),
            pl.BlockSpec((_K, _D), lambda b: (0, 0)),
        ],
        out_specs=pl.BlockSpec((1, 1, _HW), lambda b: (b, 0, 0)),
        out_shape=jax.ShapeDtypeStruct((_B, 1, _HW), jnp.int32),
        compiler_params=pltpu.CompilerParams(
            dimension_semantics=("parallel",)),
    )(zrh, zr, W_h, W_emb)


_GATHER_WINDOW = 128


def _sc_gather(W_emb, ids_flat):
    ids2 = ids_flat.reshape(1, _K)
    mesh = plsc.VectorSubcoreMesh(core_axis_name="core",
                                  subcore_axis_name="subcore")

    @pl.kernel(out_type=jax.ShapeDtypeStruct((_K, _D), W_emb.dtype),
               mesh=mesh)
    def gather_kernel(w_hbm, i_hbm, o_hbm):
        def body(i_vmem, o_vmem):
            pltpu.sync_copy(w_hbm.at[i_vmem.at[0]], o_vmem)

        pltpu.emit_pipeline(
            body,
            grid=(_K // _GATHER_WINDOW,),
            in_specs=[pl.BlockSpec((1, _GATHER_WINDOW),
                                   index_map=lambda i: (0, i))],
            out_specs=[pl.BlockSpec((_GATHER_WINDOW, _D),
                                    index_map=lambda i: (i, 0))],
            core_axis_name=("core", "subcore"),
            dimension_semantics=(pltpu.PARALLEL,),
        )(i_hbm, o_hbm)

    return gather_kernel(W_emb, ids2)


def kernel(z, W_emb):
    b, d, h, w = z.shape
    zr = z.reshape(b, d, h * w)
    ids = _token_ids(zr.astype(jnp.bfloat16), zr,
                     W_emb.astype(jnp.bfloat16), W_emb)
    rows = _sc_gather(W_emb, ids.reshape(b * h * w))
    out = rows.reshape(b, h * w, d).transpose(0, 2, 1)
    return out.reshape(b, d, h, w)
